# baseline (speedup 1.0000x reference)
import functools

import jax
import jax.numpy as jnp
from jax import lax
from jax.experimental import pallas as pl
from jax.experimental.pallas import tpu as pltpu
from jax.experimental.pallas import tpu_sc as plsc

_N = 1000
_D = 32
_E = 100
_EPAD = 128
_EPW = 8
_W = 128

_mesh = plsc.VectorSubcoreMesh(core_axis_name="c", subcore_axis_name="s",
                               num_cores=2, num_subcores=16)


@functools.partial(
    pl.kernel,
    out_type=jax.ShapeDtypeStruct((2 * _EPAD, _W), jnp.float32),
    mesh=_mesh,
    scratch_types=[
        pltpu.VMEM((_EPW,), jnp.int32),
        pltpu.VMEM((_EPW, _W), jnp.float32),
        pltpu.SemaphoreType.DMA,
    ],
)
def _sc_gather(feats, edges, out, src_v, rows_v, sem):
    c = lax.axis_index("c")
    s = lax.axis_index("s")
    ebase = c * _EPAD + s * _EPW
    pltpu.sync_copy(edges.at[pl.ds(ebase, _EPW)], src_v)
    pltpu.async_copy(feats.at[src_v], rows_v, sem).wait()
    pltpu.sync_copy(rows_v, out.at[pl.ds(ebase, _EPW)])


def _dott(a, bt):
    return jax.lax.dot_general(a, bt, (((1,), (1,)), ((), ())),
                               preferred_element_type=jnp.float32)


def _dotx(a, b):
    return jax.lax.dot(a, b, precision=jax.lax.Precision.HIGHEST,
                       preferred_element_type=jnp.float32)


def _tc1_body(m_ref, xp_ref, ed_ref,
              w1pl_ref, w1pr_ref, b1p_ref, w1nl_ref, w1nr_ref, b1n_ref,
              z_ref, inv_ref):
    f32 = jnp.float32
    x = xp_ref[0:_N, 0:_D]
    iota = lax.broadcasted_iota(jnp.int32, (_N, _EPAD), 0)
    mp = (iota == ed_ref[2:3, :]).astype(f32)
    mn = (iota == ed_ref[3:4, :]).astype(f32)
    ip = 1.0 / jnp.maximum(jnp.sum(mp, axis=1, keepdims=True), 1.0)
    im = 1.0 / jnp.maximum(jnp.sum(mn, axis=1, keepdims=True), 1.0)
    aggp = _dotx(mp, m_ref[0:_EPAD, 0:_D]) * ip
    aggn = _dotx(mn, m_ref[_EPAD:2 * _EPAD, 0:_D]) * im
    h = jnp.concatenate([aggp, aggn, x], axis=-1)
    zdd = jnp.zeros((_D, _D), f32)
    w1t = jnp.concatenate([
        jnp.concatenate([w1pl_ref[...], zdd], axis=0),
        jnp.concatenate([zdd, w1nl_ref[...]], axis=0),
        jnp.concatenate([w1pr_ref[...], w1nr_ref[...]], axis=0),
    ], axis=1)
    b1 = jnp.concatenate([b1p_ref[...], b1n_ref[...]], axis=-1)
    z = jnp.maximum(_dott(h, w1t) + b1, 0.0)
    z_ref[...] = jnp.concatenate(
        [z, jnp.zeros((_N, _W - 2 * _D), f32)], axis=-1)
    inv_ref[...] = jnp.concatenate([ip, im], axis=-1)


def _tc2_body(m_ref, z_ref, inv_ref, ed_ref,
              w2pl_ref, w2pr_ref, b2p_ref, w2nl_ref, w2nr_ref, b2n_ref,
              out_ref):
    f32 = jnp.float32
    z = z_ref[0:_N, 0:2 * _D]
    ip = inv_ref[:, 0:1]
    im = inv_ref[:, 1:2]
    iota = lax.broadcasted_iota(jnp.int32, (_N, _EPAD), 0)
    mp = (iota == ed_ref[2:3, :]).astype(f32)
    mn = (iota == ed_ref[3:4, :]).astype(f32)
    bp = _dotx(mp, m_ref[0:_EPAD, 0:2 * _D]) * ip
    bn = _dotx(mn, m_ref[_EPAD:2 * _EPAD, 0:2 * _D]) * im
    h = jnp.concatenate([bp, bn, z], axis=-1)
    zdd = jnp.zeros((_D, _D), f32)
    w2pl = w2pl_ref[...]
    w2nl = w2nl_ref[...]
    w2t = jnp.concatenate([
        jnp.concatenate([w2pl[:, 0:_D], zdd], axis=0),
        jnp.concatenate([zdd, w2nl[:, 0:_D]], axis=0),
        jnp.concatenate([zdd, w2nl[:, _D:]], axis=0),
        jnp.concatenate([w2pl[:, _D:], zdd], axis=0),
        jnp.concatenate([w2pr_ref[...], zdd], axis=0),
        jnp.concatenate([zdd, w2nr_ref[...]], axis=0),
    ], axis=1)
    b2 = jnp.concatenate([b2p_ref[...], b2n_ref[...]], axis=-1)
    out_ref[...] = jnp.maximum(_dott(h, w2t) + b2, 0.0)


def kernel(x, pos_edge_index, neg_edge_index,
           w1_pos_l, w1_pos_r, b1_pos_r,
           w1_neg_l, w1_neg_r, b1_neg_r,
           w2_pos_l, w2_pos_r, b2_pos_r,
           w2_neg_l, w2_neg_r, b2_neg_r):
    f32 = jnp.float32
    pe = pos_edge_index.astype(jnp.int32)
    ne = neg_edge_index.astype(jnp.int32)
    pad_s = jnp.zeros((_EPAD - _E,), jnp.int32)
    pad_d = jnp.full((_EPAD - _E,), _N, jnp.int32)
    edges = jnp.concatenate([pe[0], pad_s, ne[0], pad_s,
                             pe[1], pad_d, ne[1], pad_d])
    edges2d = edges.reshape(4, _EPAD)
    x_pad = jnp.pad(x, ((0, 0), (0, _W - _D)))
    b1p = b1_pos_r.reshape(1, _D)
    b1n = b1_neg_r.reshape(1, _D)
    b2p = b2_pos_r.reshape(1, _D)
    b2n = b2_neg_r.reshape(1, _D)

    m1 = _sc_gather(x_pad, edges)
    z_pad, inv = pl.pallas_call(
        _tc1_body,
        out_shape=(jax.ShapeDtypeStruct((_N, _W), f32),
                   jax.ShapeDtypeStruct((_N, 2), f32)),
    )(m1, x_pad, edges2d,
      w1_pos_l, w1_pos_r, b1p, w1_neg_l, w1_neg_r, b1n)
    m2 = _sc_gather(z_pad, edges)
    return pl.pallas_call(
        _tc2_body,
        out_shape=jax.ShapeDtypeStruct((_N, 2 * _D), f32),
    )(m2, z_pad, inv, edges2d, w2_pos_l, w2_pos_r, b2p, w2_neg_l, w2_neg_r, b2n)


# speedup vs baseline: 1.0118x; 1.0118x over previous
import functools

import jax
import jax.numpy as jnp
from jax import lax
from jax.experimental import pallas as pl
from jax.experimental.pallas import tpu as pltpu
from jax.experimental.pallas import tpu_sc as plsc

_N = 1000
_D = 32
_E = 100
_EPAD = 128
_ROWS = 1024
_EPW = 8
_W = 128

_mesh = plsc.VectorSubcoreMesh(core_axis_name="c", subcore_axis_name="s",
                               num_cores=2, num_subcores=16)


@functools.partial(
    pl.kernel,
    out_type=(jax.ShapeDtypeStruct((_ROWS, _W), jnp.float32),
              jax.ShapeDtypeStruct((_ROWS, _W), jnp.float32)),
    mesh=_mesh,
    scratch_types=[
        pltpu.VMEM((_EPW,), jnp.int32),
        pltpu.VMEM((_EPW,), jnp.int32),
        pltpu.VMEM((_EPW, _W), jnp.float32),
        pltpu.VMEM((_EPW, _W), jnp.float32),
        pltpu.VMEM_SHARED((_ROWS, _W), jnp.float32),
        pltpu.SemaphoreType.DMA,
        pltpu.SemaphoreType.DMA,
        pltpu.SemaphoreType.DMA,
        pltpu.SemaphoreType.DMA,
    ],
)
def _sc_aggr(feats, edges, outp, outn,
             src_v, dst_v, z8, rows_v, acc, sema, semb, semc, semd):
    c = lax.axis_index("c")
    s = lax.axis_index("s")
    ebase = c * _EPAD + s * _EPW
    cpd = pltpu.async_copy(edges.at[pl.ds(2 * _EPAD + ebase, _EPW)], dst_v, semb)
    cps = pltpu.async_copy(edges.at[pl.ds(ebase, _EPW)], src_v, sema)

    def _zrow(i, carry):
        for j in range(_W // 16):
            z8[i, pl.ds(16 * j, 16)] = jnp.zeros((16,), jnp.float32)
        return carry

    lax.fori_loop(0, _EPW, _zrow, 0)
    cpd.wait()
    zs = pltpu.async_copy(z8, acc.at[dst_v], semd)
    cps.wait()
    pltpu.async_copy(feats.at[src_v], rows_v, semc).wait()
    zs.wait()

    plsc.subcore_barrier()
    pltpu.sync_copy(rows_v, acc.at[dst_v], add=True)
    plsc.subcore_barrier()

    pltpu.async_copy(acc.at[dst_v], rows_v, semc).wait()

    @pl.when(c == 0)
    def _ep():
        pltpu.sync_copy(rows_v, outp.at[dst_v])

    @pl.when(c == 1)
    def _en():
        pltpu.sync_copy(rows_v, outn.at[dst_v])


def _dott(a, bt):
    return jax.lax.dot_general(a, bt, (((1,), (1,)), ((), ())),
                               preferred_element_type=jnp.float32)


def _tc1_body(sp_ref, sn_ref, xp_ref, ed_ref,
              w1pl_ref, w1pr_ref, b1p_ref, w1nl_ref, w1nr_ref, b1n_ref,
              z_ref, inv_ref):
    f32 = jnp.float32
    x = xp_ref[0:_N, 0:_D]
    psum = sp_ref[0:_N, 0:_D]
    nsum = sn_ref[0:_N, 0:_D]
    iota = lax.broadcasted_iota(jnp.int32, (_N, _EPAD), 0)
    cp = jnp.sum((iota == ed_ref[2:3, :]).astype(f32), axis=1, keepdims=True)
    cn = jnp.sum((iota == ed_ref[3:4, :]).astype(f32), axis=1, keepdims=True)
    ip = 1.0 / jnp.maximum(cp, 1.0)
    im = 1.0 / jnp.maximum(cn, 1.0)
    aggp = jnp.where(cp > 0.0, psum * ip, 0.0)
    aggn = jnp.where(cn > 0.0, nsum * im, 0.0)
    h = jnp.concatenate([aggp, aggn, x], axis=-1)
    zdd = jnp.zeros((_D, _D), f32)
    w1t = jnp.concatenate([
        jnp.concatenate([w1pl_ref[...], zdd], axis=0),
        jnp.concatenate([zdd, w1nl_ref[...]], axis=0),
        jnp.concatenate([w1pr_ref[...], w1nr_ref[...]], axis=0),
    ], axis=1)
    b1 = jnp.concatenate([b1p_ref[...], b1n_ref[...]], axis=-1)
    z = jnp.maximum(_dott(h, w1t) + b1, 0.0)
    z_ref[...] = jnp.concatenate(
        [z, jnp.zeros((_N, _W - 2 * _D), f32)], axis=-1)
    inv_ref[...] = jnp.concatenate([jnp.where(cp > 0.0, ip, 0.0),
                                    jnp.where(cn > 0.0, im, 0.0)], axis=-1)


def _tc2_body(sp_ref, sn_ref, z_ref, inv_ref,
              w2pl_ref, w2pr_ref, b2p_ref, w2nl_ref, w2nr_ref, b2n_ref,
              out_ref):
    f32 = jnp.float32
    z = z_ref[0:_N, 0:2 * _D]
    ip = inv_ref[:, 0:1]
    im = inv_ref[:, 1:2]
    bp = jnp.where(ip > 0.0, sp_ref[0:_N, 0:2 * _D] * ip, 0.0)
    bn = jnp.where(im > 0.0, sn_ref[0:_N, 0:2 * _D] * im, 0.0)
    h = jnp.concatenate([bp, bn, z], axis=-1)
    zdd = jnp.zeros((_D, _D), f32)
    w2pl = w2pl_ref[...]
    w2nl = w2nl_ref[...]
    w2t = jnp.concatenate([
        jnp.concatenate([w2pl[:, 0:_D], zdd], axis=0),
        jnp.concatenate([zdd, w2nl[:, 0:_D]], axis=0),
        jnp.concatenate([zdd, w2nl[:, _D:]], axis=0),
        jnp.concatenate([w2pl[:, _D:], zdd], axis=0),
        jnp.concatenate([w2pr_ref[...], zdd], axis=0),
        jnp.concatenate([zdd, w2nr_ref[...]], axis=0),
    ], axis=1)
    b2 = jnp.concatenate([b2p_ref[...], b2n_ref[...]], axis=-1)
    out_ref[...] = jnp.maximum(_dott(h, w2t) + b2, 0.0)


def kernel(x, pos_edge_index, neg_edge_index,
           w1_pos_l, w1_pos_r, b1_pos_r,
           w1_neg_l, w1_neg_r, b1_neg_r,
           w2_pos_l, w2_pos_r, b2_pos_r,
           w2_neg_l, w2_neg_r, b2_neg_r):
    f32 = jnp.float32
    pe = pos_edge_index.astype(jnp.int32)
    ne = neg_edge_index.astype(jnp.int32)
    pad_s = jnp.zeros((_EPAD - _E,), jnp.int32)
    pad_d = jnp.full((_EPAD - _E,), _N, jnp.int32)
    edges = jnp.concatenate([pe[0], pad_s, ne[0], pad_s,
                             pe[1], pad_d, ne[1], pad_d])
    edges2d = edges.reshape(4, _EPAD)
    x_pad = jnp.pad(x, ((0, 0), (0, _W - _D)))
    b1p = b1_pos_r.reshape(1, _D)
    b1n = b1_neg_r.reshape(1, _D)
    b2p = b2_pos_r.reshape(1, _D)
    b2n = b2_neg_r.reshape(1, _D)

    s1p, s1n = _sc_aggr(x_pad, edges)
    z_pad, inv = pl.pallas_call(
        _tc1_body,
        out_shape=(jax.ShapeDtypeStruct((_N, _W), f32),
                   jax.ShapeDtypeStruct((_N, 2), f32)),
    )(s1p, s1n, x_pad, edges2d,
      w1_pos_l, w1_pos_r, b1p, w1_neg_l, w1_neg_r, b1n)
    s2p, s2n = _sc_aggr(z_pad, edges)
    return pl.pallas_call(
        _tc2_body,
        out_shape=jax.ShapeDtypeStruct((_N, 2 * _D), f32),
    )(s2p, s2n, z_pad, inv, w2_pos_l, w2_pos_r, b2p, w2_neg_l, w2_neg_r, b2n)
